# baseline (speedup 1.0000x reference)
import jax
import jax.numpy as jnp
from jax import lax
from jax.experimental import pallas as pl
from jax.experimental.pallas import tpu as pltpu
from jax.experimental.pallas import tpu_sc as plsc

NC, NS, LANES = 2, 16, 16
NW = NC * NS

MAGIC = 12582912.0
MAGIC16 = 49152.0
REP = 8
TPAD = 32784

ROWS = 4 * 4096
COLS = 2048
PER_W_ROWS = ROWS // NW
CHUNK_ROWS = 8
N_CHUNKS = PER_W_ROWS // CHUNK_ROWS
VREGS_PER_ROW = COLS // LANES


def _compute_chunk(xbuf, ybuf, table_v):
    @plsc.parallel_loop(0, CHUNK_ROWS * VREGS_PER_ROW, unroll=16)
    def _vreg(i):
            r = (i >> 3) & (CHUNK_ROWS - 1)
            col = ((i >> 6) << 7) + ((i & 7) << 4)
            xv = xbuf[r, pl.ds(col, LANES)]
            a = xv * 256.0 + MAGIC
            c = jnp.maximum(jnp.minimum(a, MAGIC + 32767.0), MAGIC - 32768.0)
            xq = c - MAGIC
            e = jnp.maximum(jnp.minimum(c, MAGIC + 2048.0), MAGIC - 2048.0)
            idx = (e - (MAGIC - 2048.0)).astype(jnp.int32)
            lane = lax.iota(jnp.int32, LANES) & (REP - 1)
            idx = (idx << 3) + lane
            sv = plsc.load_gather(table_v, [idx])
            p = xq * sv
            ybuf[r, pl.ds(col, LANES)] = (p + MAGIC16) - MAGIC16


def _sc_body(x_hbm, table_hbm, out_hbm,
             table_v, xb0, xb1, yb0, yb1, si0, si1, so0, so1):
    wid = lax.axis_index("s") * NC + lax.axis_index("c")
    pltpu.sync_copy(table_hbm, table_v)
    base = wid * PER_W_ROWS
    xbufs, ybufs = (xb0, xb1), (yb0, yb1)
    sins, souts = (si0, si1), (so0, so1)

    pltpu.async_copy(x_hbm.at[pl.ds(base, CHUNK_ROWS), :], xb0, si0)
    pltpu.async_copy(x_hbm.at[pl.ds(base + CHUNK_ROWS, CHUNK_ROWS), :], xb1, si1)

    @pl.loop(0, N_CHUNKS, step=2)
    def _outer(g0):
        for b in range(2):
            c = g0 + b
            off = base + c * CHUNK_ROWS
            pltpu.make_async_copy(
                x_hbm.at[pl.ds(off, CHUNK_ROWS), :], xbufs[b], sins[b]).wait()

            @pl.when(c >= 2)
            def _():
                pltpu.make_async_copy(
                    ybufs[b], out_hbm.at[pl.ds(off, CHUNK_ROWS), :],
                    souts[b]).wait()

            _compute_chunk(xbufs[b], ybufs[b], table_v)
            pltpu.async_copy(ybufs[b], out_hbm.at[pl.ds(off, CHUNK_ROWS), :],
                             souts[b])

            @pl.when(c + 2 < N_CHUNKS)
            def _():
                pltpu.async_copy(
                    x_hbm.at[pl.ds(off + 2 * CHUNK_ROWS, CHUNK_ROWS), :],
                    xbufs[b], sins[b])

    for b in range(2):
        pltpu.make_async_copy(
            ybufs[b], out_hbm.at[pl.ds(base, CHUNK_ROWS), :], souts[b]).wait()


def kernel(x, table):
    b, s, d = x.shape
    x2 = x.reshape(b * s, d)
    tf = (table.astype(jnp.float32) * (1.0 / 65536.0))
    tf = jnp.repeat(tf, REP)
    tf = jnp.pad(tf, (0, TPAD - tf.shape[0]))
    mesh = plsc.VectorSubcoreMesh(core_axis_name="c", subcore_axis_name="s")
    run = pl.kernel(
        _sc_body,
        out_type=jax.ShapeDtypeStruct((b * s, d), jnp.float32),
        mesh=mesh,
        compiler_params=pltpu.CompilerParams(
            needs_layout_passes=False,
            use_tc_tiling_on_sc=True,
        ),
        scratch_types=[
            pltpu.VMEM((TPAD,), jnp.float32),
            pltpu.VMEM((CHUNK_ROWS, COLS), jnp.float32),
            pltpu.VMEM((CHUNK_ROWS, COLS), jnp.float32),
            pltpu.VMEM((CHUNK_ROWS, COLS), jnp.float32),
            pltpu.VMEM((CHUNK_ROWS, COLS), jnp.float32),
            pltpu.SemaphoreType.DMA,
            pltpu.SemaphoreType.DMA,
            pltpu.SemaphoreType.DMA,
            pltpu.SemaphoreType.DMA,
        ],
    )
    return run(x2, tf).reshape(b, s, d)


# speedup vs baseline: 1.1344x; 1.1344x over previous
import jax
import jax.numpy as jnp
from jax import lax
from jax.experimental import pallas as pl
from jax.experimental.pallas import tpu as pltpu
from jax.experimental.pallas import tpu_sc as plsc

NC, NS, LANES = 2, 16, 16
NW = NC * NS

MAGIC = 12582912.0
MAGIC16 = 49152.0
TPAD = 4160

ROWS = 4 * 4096
COLS = 2048
PER_W_ROWS = ROWS // NW
CHUNK_ROWS = 8
N_CHUNKS = PER_W_ROWS // CHUNK_ROWS
VREGS_PER_ROW = COLS // LANES


def _compute_chunk(xbuf, ybuf, table_v):
    @plsc.parallel_loop(0, CHUNK_ROWS * VREGS_PER_ROW, unroll=16)
    def _vreg(i):
            r = (i >> 3) & (CHUNK_ROWS - 1)
            col = ((i >> 6) << 7) + ((i & 7) << 4)
            xv = xbuf[r, pl.ds(col, LANES)]
            a = xv * 256.0 + MAGIC
            c = jnp.maximum(jnp.minimum(a, MAGIC + 32767.0), MAGIC - 32768.0)
            xq = c - MAGIC
            e = jnp.maximum(jnp.minimum(c, MAGIC + 2048.0), MAGIC - 2048.0)
            idx = (e - (MAGIC - 2048.0)).astype(jnp.int32)
            sv = plsc.load_gather(table_v, [idx])
            p = xq * sv
            ybuf[r, pl.ds(col, LANES)] = (p + MAGIC16) - MAGIC16


def _sc_body(x_hbm, table_hbm, out_hbm,
             table_v, xb0, xb1, yb0, yb1, si0, si1, so0, so1):
    wid = lax.axis_index("s") * NC + lax.axis_index("c")
    pltpu.sync_copy(table_hbm, table_v)
    base = wid * PER_W_ROWS
    xbufs, ybufs = (xb0, xb1), (yb0, yb1)
    sins, souts = (si0, si1), (so0, so1)

    pltpu.async_copy(x_hbm.at[pl.ds(base, CHUNK_ROWS), :], xb0, si0)
    pltpu.async_copy(x_hbm.at[pl.ds(base + CHUNK_ROWS, CHUNK_ROWS), :], xb1, si1)

    @pl.loop(0, N_CHUNKS, step=2)
    def _outer(g0):
        for b in range(2):
            c = g0 + b
            off = base + c * CHUNK_ROWS
            pltpu.make_async_copy(
                x_hbm.at[pl.ds(off, CHUNK_ROWS), :], xbufs[b], sins[b]).wait()

            @pl.when(c >= 2)
            def _():
                pltpu.make_async_copy(
                    ybufs[b], out_hbm.at[pl.ds(off, CHUNK_ROWS), :],
                    souts[b]).wait()

            _compute_chunk(xbufs[b], ybufs[b], table_v)
            pltpu.async_copy(ybufs[b], out_hbm.at[pl.ds(off, CHUNK_ROWS), :],
                             souts[b])

            @pl.when(c + 2 < N_CHUNKS)
            def _():
                pltpu.async_copy(
                    x_hbm.at[pl.ds(off + 2 * CHUNK_ROWS, CHUNK_ROWS), :],
                    xbufs[b], sins[b])

    for b in range(2):
        pltpu.make_async_copy(
            ybufs[b], out_hbm.at[pl.ds(base, CHUNK_ROWS), :], souts[b]).wait()


def kernel(x, table):
    b, s, d = x.shape
    x2 = x.reshape(b * s, d)
    tf = (table.astype(jnp.float32) * (1.0 / 65536.0))
    tf = jnp.pad(tf, (0, TPAD - tf.shape[0]))
    mesh = plsc.VectorSubcoreMesh(core_axis_name="c", subcore_axis_name="s")
    run = pl.kernel(
        _sc_body,
        out_type=jax.ShapeDtypeStruct((b * s, d), jnp.float32),
        mesh=mesh,
        compiler_params=pltpu.CompilerParams(
            needs_layout_passes=False,
            use_tc_tiling_on_sc=True,
        ),
        scratch_types=[
            pltpu.VMEM((TPAD,), jnp.float32),
            pltpu.VMEM((CHUNK_ROWS, COLS), jnp.float32),
            pltpu.VMEM((CHUNK_ROWS, COLS), jnp.float32),
            pltpu.VMEM((CHUNK_ROWS, COLS), jnp.float32),
            pltpu.VMEM((CHUNK_ROWS, COLS), jnp.float32),
            pltpu.SemaphoreType.DMA,
            pltpu.SemaphoreType.DMA,
            pltpu.SemaphoreType.DMA,
            pltpu.SemaphoreType.DMA,
        ],
    )
    return run(x2, tf).reshape(b, s, d)
